# baseline (speedup 1.0000x reference)
import functools

import jax
import jax.numpy as jnp
from jax.experimental import pallas as pl
from jax.experimental.pallas import tpu as pltpu

M = 8192
K = 4096
N = 4096
E = 4

BT = 512
BC = 2048
CBLKS = N // BC
TBLKS = M // BT


def _router_kernel(x_ref, w1_ref, b1_ref, w2_ref, b2_ref,
                   rw_ref, ad_ref, dm_ref, acc_ref):
    j = pl.program_id(0)
    i = pl.program_id(1)

    part = jax.lax.dot_general(
        x_ref[...], w1_ref[...],
        (((1,), (0,)), ((), ())),
        preferred_element_type=jnp.float32,
    )
    pre = part + b1_ref[0, :]
    h = 0.5 * pre * (1.0 + jax.lax.erf(pre * 0.7071067811865476))
    lp = jax.lax.dot_general(
        h, w2_ref[...],
        (((1,), (0,)), ((), ())),
        preferred_element_type=jnp.float32,
    )
    lpt = lp.T

    base = i * BT

    @pl.when(j == 0)
    def _init():
        acc_ref[:, pl.ds(base, BT)] = lpt

    @pl.when(j > 0)
    def _accum():
        acc_ref[:, pl.ds(base, BT)] += lpt

    @pl.when(j == CBLKS - 1)
    def _finalize():
        logits_t = acc_ref[:, pl.ds(base, BT)] + b2_ref[:, :1]
        m = jnp.max(logits_t, axis=0, keepdims=True)
        e = jnp.exp(logits_t - m)
        probs_t = e / jnp.sum(e, axis=0, keepdims=True)
        rw_ref[...] = probs_t.T
        amax = jnp.argmax(logits_t, axis=0).astype(jnp.int32)
        ad_ref[...] = amax[:, None]
        lane = jax.lax.broadcasted_iota(jnp.int32, (BT, E), 1)
        dm_ref[...] = (lane == amax[:, None]).astype(jnp.int32)


@functools.partial(jax.jit, static_argnums=())
def _run(x, W1, b1r, W2, b2r):
    grid = (CBLKS, TBLKS)
    rw, ad, dm = pl.pallas_call(
        _router_kernel,
        grid=grid,
        in_specs=[
            pl.BlockSpec((BT, K), lambda j, i: (i, 0)),
            pl.BlockSpec((K, BC), lambda j, i: (0, j),
                         pipeline_mode=pl.Buffered(buffer_count=1)),
            pl.BlockSpec((1, BC), lambda j, i: (0, j)),
            pl.BlockSpec((BC, E), lambda j, i: (j, 0)),
            pl.BlockSpec((E, 1), lambda j, i: (0, 0)),
        ],
        out_specs=[
            pl.BlockSpec((BT, E), lambda j, i: (i, 0)),
            pl.BlockSpec((BT, 1), lambda j, i: (i, 0)),
            pl.BlockSpec((BT, E), lambda j, i: (i, 0)),
        ],
        out_shape=[
            jax.ShapeDtypeStruct((M, E), jnp.float32),
            jax.ShapeDtypeStruct((M, 1), jnp.int32),
            jax.ShapeDtypeStruct((M, E), jnp.int32),
        ],
        scratch_shapes=[pltpu.VMEM((E, M), jnp.float32)],
        compiler_params=pltpu.CompilerParams(
            dimension_semantics=("arbitrary", "parallel")),
    )(x, W1, b1r, W2, b2r)
    return rw, ad, dm


def kernel(hidden_states, W1, b1, W2, b2):
    B, S, H = hidden_states.shape
    x = hidden_states.reshape(B * S, H)
    rw, ad, dm = _run(x, W1, b1.reshape(1, -1), W2, b2.reshape(-1, 1))
    router_weights = rw.reshape(B, S, E)
    assigned_depth = ad.reshape(B, S)
    depth_mask = dm.astype(jnp.bool_).reshape(B, S, E)
    return (router_weights, assigned_depth, depth_mask)


# speedup vs baseline: 1.0108x; 1.0108x over previous
import functools

import jax
import jax.numpy as jnp
from jax.experimental import pallas as pl
from jax.experimental.pallas import tpu as pltpu

M = 8192
K = 4096
N = 4096
E = 4

BT = 512
BC = 2048
CBLKS = N // BC
TBLKS = M // BT


def _router_kernel(x_ref, w1_ref, b1_ref, w2_ref, b2_ref,
                   rw_ref, ad_ref, dm_ref, acc_ref):
    j = pl.program_id(0)
    i = pl.program_id(1)

    part = jax.lax.dot_general(
        x_ref[...], w1_ref[...],
        (((1,), (0,)), ((), ())),
        preferred_element_type=jnp.float32,
    )
    pre = part + b1_ref[0, :]
    h = 0.5 * pre * (1.0 + jax.lax.erf(pre * 0.7071067811865476))
    lp = jax.lax.dot_general(
        h, w2_ref[...],
        (((1,), (0,)), ((), ())),
        preferred_element_type=jnp.float32,
    )
    lpt = lp.T

    base = i * BT

    @pl.when(j == 0)
    def _init():
        acc_ref[:, pl.ds(base, BT)] = lpt

    @pl.when(j > 0)
    def _accum():
        acc_ref[:, pl.ds(base, BT)] += lpt

    @pl.when(j == CBLKS - 1)
    def _finalize():
        logits_t = acc_ref[:, pl.ds(base, BT)] + b2_ref[:, :1]
        m = jnp.max(logits_t, axis=0, keepdims=True)
        e = jnp.exp(logits_t - m)
        probs_t = e / jnp.sum(e, axis=0, keepdims=True)
        rw_ref[0, :, :] = probs_t.T
        amax = jnp.argmax(logits_t, axis=0).astype(jnp.int32)
        ad_ref[0, 0, :] = amax
        lane = jax.lax.broadcasted_iota(jnp.int32, (BT, E), 1)
        dm_ref[0, :, :] = lane == amax[:, None]


@functools.partial(jax.jit, static_argnums=())
def _run(x, W1, b1r, W2, b2r):
    grid = (CBLKS, TBLKS)
    rw, ad, dm = pl.pallas_call(
        _router_kernel,
        grid=grid,
        in_specs=[
            pl.BlockSpec((BT, K), lambda j, i: (i, 0)),
            pl.BlockSpec((K, BC), lambda j, i: (0, j),
                         pipeline_mode=pl.Buffered(buffer_count=1)),
            pl.BlockSpec((1, BC), lambda j, i: (0, j)),
            pl.BlockSpec((BC, E), lambda j, i: (j, 0)),
            pl.BlockSpec((E, 1), lambda j, i: (0, 0)),
        ],
        out_specs=[
            pl.BlockSpec((1, BT, E), lambda j, i: (i, 0, 0)),
            pl.BlockSpec((1, 1, BT), lambda j, i: (i, 0, 0)),
            pl.BlockSpec((1, BT, E), lambda j, i: (i, 0, 0)),
        ],
        out_shape=[
            jax.ShapeDtypeStruct((M // BT, BT, E), jnp.float32),
            jax.ShapeDtypeStruct((M // BT, 1, BT), jnp.int32),
            jax.ShapeDtypeStruct((M // BT, BT, E), jnp.bool_),
        ],
        scratch_shapes=[pltpu.VMEM((E, M), jnp.float32)],
        compiler_params=pltpu.CompilerParams(
            dimension_semantics=("arbitrary", "parallel")),
    )(x, W1, b1r, W2, b2r)
    return rw, ad, dm


def kernel(hidden_states, W1, b1, W2, b2):
    B, S, H = hidden_states.shape
    x = hidden_states.reshape(B * S, H)
    rw, ad, dm = _run(x, W1, b1.reshape(1, -1), W2, b2.reshape(-1, 1))
    router_weights = rw.reshape(B, S, E)
    assigned_depth = ad.reshape(B, S)
    depth_mask = dm.reshape(B, S, E)
    return (router_weights, assigned_depth, depth_mask)
